# baseline (speedup 1.0000x reference)
import functools

import jax
import jax.numpy as jnp
from jax import lax
from jax.experimental import pallas as pl
from jax.experimental.pallas import tpu as pltpu
from jax.experimental.pallas import tpu_sc as plsc

B = 4096
L = 50
EMBED_DIM = 128
NUM_CORES = 2
NUM_SUBCORES = 16
NUM_WORKERS = NUM_CORES * NUM_SUBCORES
B_PER_W = B // NUM_WORKERS
WINDOW = 16
NV = EMBED_DIM // 16


def _sc_body(idx_hbm, table_hbm, out_hbm, idx_all, acc, sem):
    wid = lax.axis_index("s") * NUM_CORES + lax.axis_index("c")
    out_base = wid * B_PER_W

    pltpu.sync_copy(idx_hbm.at[pl.ds(wid * (B_PER_W * L), B_PER_W * L)],
                    idx_all)

    def zero_body(r, carry):
        for d in range(NV):
            acc[r, pl.ds(d * 16, 16)] = jnp.zeros((16,), jnp.float32)
        return carry

    lax.fori_loop(0, B_PER_W, zero_body, 0)

    def issue(l):
        pltpu.async_copy(table_hbm.at[idx_all.at[pl.ds(l * B_PER_W, B_PER_W)]], acc, sem, add=True)

    def wait_one():
        pltpu.make_async_copy(table_hbm.at[idx_all.at[pl.ds(0, B_PER_W)]], acc, sem).wait()

    def fire_body(l, carry):
        issue(l)

        @pl.when(l >= WINDOW)
        def _():
            wait_one()
        return carry

    lax.fori_loop(0, L, fire_body, 0)

    def drain_body(i, carry):
        wait_one()
        return carry

    lax.fori_loop(0, WINDOW, drain_body, 0)
    pltpu.sync_copy(acc, out_hbm.at[pl.ds(out_base, B_PER_W)])


@jax.jit
def kernel(indices, table):
    idx_t = (indices.astype(jnp.int32)
             .reshape(NUM_WORKERS, B_PER_W, L)
             .transpose(0, 2, 1)
             .reshape(-1))
    run = pl.kernel(
        _sc_body,
        out_type=jax.ShapeDtypeStruct((B, EMBED_DIM), jnp.float32),
        mesh=plsc.VectorSubcoreMesh(core_axis_name="c", subcore_axis_name="s"),
        scratch_types=[
            pltpu.VMEM((L * B_PER_W,), jnp.int32),
            pltpu.VMEM((B_PER_W, EMBED_DIM), jnp.float32),
            pltpu.SemaphoreType.DMA,
        ],
    )
    return run(idx_t, table)


# speedup vs baseline: 1.0168x; 1.0168x over previous
import functools

import jax
import jax.numpy as jnp
from jax import lax
from jax.experimental import pallas as pl
from jax.experimental.pallas import tpu as pltpu
from jax.experimental.pallas import tpu_sc as plsc

B = 4096
L = 50
EMBED_DIM = 128
NUM_CORES = 2
NUM_SUBCORES = 16
NUM_WORKERS = NUM_CORES * NUM_SUBCORES
B_PER_W = B // NUM_WORKERS
HALF = B_PER_W // 2
WINDOW = 8
NV = EMBED_DIM // 16


def _sc_body(idx_hbm, table_hbm, out_hbm, idx_all, acc0, acc1,
             isem, sem0, sem1, osem):
    wid = lax.axis_index("s") * NUM_CORES + lax.axis_index("c")
    out_base = wid * B_PER_W

    idx_cp = pltpu.async_copy(
        idx_hbm.at[pl.ds(wid * (B_PER_W * L), B_PER_W * L)], idx_all, isem)

    def zero_body(r, carry):
        for d in range(NV):
            acc0[r, pl.ds(d * 16, 16)] = jnp.zeros((16,), jnp.float32)
            acc1[r, pl.ds(d * 16, 16)] = jnp.zeros((16,), jnp.float32)
        return carry

    lax.fori_loop(0, HALF, zero_body, 0)
    idx_cp.wait()

    halves = ((acc0, 0, sem0), (acc1, HALF, sem1))

    def issue(l, acc, off, sem):
        pltpu.async_copy(
            table_hbm.at[idx_all.at[pl.ds(l * B_PER_W + off, HALF)]],
            acc, sem, add=True)

    def wait_one(acc, sem):
        pltpu.make_async_copy(
            table_hbm.at[idx_all.at[pl.ds(0, HALF)]], acc, sem).wait()

    def fire_body(l, carry):
        for acc, off, sem in halves:
            issue(l, acc, off, sem)

            @pl.when(l >= WINDOW)
            def _():
                wait_one(acc, sem)
        return carry

    lax.fori_loop(0, L, fire_body, 0)

    def drain0_body(i, carry):
        wait_one(acc0, sem0)
        return carry

    lax.fori_loop(0, WINDOW, drain0_body, 0)
    out_cp = pltpu.async_copy(acc0, out_hbm.at[pl.ds(out_base, HALF)], osem)

    def drain1_body(i, carry):
        wait_one(acc1, sem1)
        return carry

    lax.fori_loop(0, WINDOW, drain1_body, 0)
    pltpu.sync_copy(acc1, out_hbm.at[pl.ds(out_base + HALF, HALF)])
    out_cp.wait()


@jax.jit
def kernel(indices, table):
    idx_t = (indices.astype(jnp.int32)
             .reshape(NUM_WORKERS, B_PER_W, L)
             .transpose(0, 2, 1)
             .reshape(-1))
    run = pl.kernel(
        _sc_body,
        out_type=jax.ShapeDtypeStruct((B, EMBED_DIM), jnp.float32),
        mesh=plsc.VectorSubcoreMesh(core_axis_name="c", subcore_axis_name="s"),
        scratch_types=[
            pltpu.VMEM((L * B_PER_W,), jnp.int32),
            pltpu.VMEM((HALF, EMBED_DIM), jnp.float32),
            pltpu.VMEM((HALF, EMBED_DIM), jnp.float32),
            pltpu.SemaphoreType.DMA,
            pltpu.SemaphoreType.DMA,
            pltpu.SemaphoreType.DMA,
            pltpu.SemaphoreType.DMA,
        ],
    )
    return run(idx_t, table)
